# baseline (speedup 1.0000x reference)
import functools

import jax
import jax.numpy as jnp
from jax import lax
from jax.experimental import pallas as pl
from jax.experimental.pallas import tpu as pltpu
from jax.experimental.pallas import tpu_sc as plsc

_ND = 32
_NC = 2
_NS = 16
_NW = _NC * _NS
_CH = 128


def _encode(idx_flat, table):
    rows = idx_flat.shape[0]
    rpw = rows // _NW

    mesh = plsc.VectorSubcoreMesh(core_axis_name="c", subcore_axis_name="s")

    @functools.partial(
        pl.kernel,
        out_type=jax.ShapeDtypeStruct((rows, _ND), jnp.float32),
        mesh=mesh,
        scratch_types=[
            pltpu.VMEM((rpw,), jnp.int32),
            pltpu.VMEM((rpw, _ND), jnp.float32),
            pltpu.VMEM_SHARED((16, _ND), jnp.float32),
            pltpu.SemaphoreType.DMA,
        ],
        compiler_params=pltpu.CompilerParams(
            use_tc_tiling_on_sc=False, skip_device_barrier=True
        ),
    )
    def run(idx_hbm, table_hbm, out_hbm, idx_v, rows_v, tbl_s, sem):
        wid = lax.axis_index("s") * _NC + lax.axis_index("c")

        @pl.when(lax.axis_index("s") == 0)
        def _stage_table():
            pltpu.sync_copy(table_hbm, tbl_s)

        pltpu.sync_copy(idx_hbm.at[pl.ds(wid * rpw, rpw)], idx_v)
        plsc.subcore_barrier()
        pltpu.async_copy(tbl_s.at[idx_v], rows_v, sem).wait()
        pltpu.sync_copy(rows_v, out_hbm.at[pl.ds(wid * rpw, rpw)])

    return run(idx_flat, table)


def kernel(addr_nibbles, nibble_basis):
    b, k = addr_nibbles.shape
    idx_flat = addr_nibbles.astype(jnp.int32).reshape(-1)
    out = _encode(idx_flat, nibble_basis)
    return out.reshape(b, k * _ND)


# speedup vs baseline: 1.0328x; 1.0328x over previous
import functools

import jax
import jax.numpy as jnp
from jax import lax
from jax.experimental import pallas as pl
from jax.experimental.pallas import tpu as pltpu
from jax.experimental.pallas import tpu_sc as plsc

_ND = 32
_NC = 2
_NS = 16
_NW = _NC * _NS
_NCHUNK = 4


def _encode(idx_flat, table):
    rows = idx_flat.shape[0]
    rpw = rows // _NW
    ch = rpw // _NCHUNK

    mesh = plsc.VectorSubcoreMesh(core_axis_name="c", subcore_axis_name="s")

    @functools.partial(
        pl.kernel,
        out_type=jax.ShapeDtypeStruct((rows, _ND), jnp.float32),
        mesh=mesh,
        scratch_types=[
            pltpu.VMEM((rpw,), jnp.int32),
            pltpu.VMEM((rpw, _ND), jnp.float32),
            pltpu.VMEM_SHARED((16, _ND), jnp.float32),
            pltpu.SemaphoreType.DMA,
            pltpu.SemaphoreType.DMA,
        ],
        compiler_params=pltpu.CompilerParams(use_tc_tiling_on_sc=False),
    )
    def run(idx_hbm, table_hbm, out_hbm, idx_v, rows_v, tbl_s, gsem, wsem):
        wid = lax.axis_index("s") * _NC + lax.axis_index("c")

        @pl.when(lax.axis_index("s") == 0)
        def _stage_table():
            pltpu.sync_copy(table_hbm, tbl_s)

        pltpu.sync_copy(idx_hbm.at[pl.ds(wid * rpw, rpw)], idx_v)
        plsc.subcore_barrier()
        gathers = [
            pltpu.async_copy(
                tbl_s.at[idx_v.at[pl.ds(j * ch, ch)]],
                rows_v.at[pl.ds(j * ch, ch)],
                gsem,
            )
            for j in range(_NCHUNK)
        ]
        writes = []
        for j in range(_NCHUNK):
            gathers[j].wait()
            writes.append(
                pltpu.async_copy(
                    rows_v.at[pl.ds(j * ch, ch)],
                    out_hbm.at[pl.ds(wid * rpw + j * ch, ch)],
                    wsem,
                )
            )
        for w in writes:
            w.wait()

    return run(idx_flat, table)


def kernel(addr_nibbles, nibble_basis):
    b, k = addr_nibbles.shape
    idx_flat = addr_nibbles.astype(jnp.int32).reshape(-1)
    out = _encode(idx_flat, nibble_basis)
    return out.reshape(b, k * _ND)


# speedup vs baseline: 1.0354x; 1.0025x over previous
import functools

import jax
import jax.numpy as jnp
from jax import lax
from jax.experimental import pallas as pl
from jax.experimental.pallas import tpu as pltpu
from jax.experimental.pallas import tpu_sc as plsc

_ND = 32
_NC = 2
_NS = 16
_NW = _NC * _NS
_CH = 128


def _encode(idx2d, table):
    rows = idx2d.shape[0] * idx2d.shape[1]
    rpw = rows // _NW
    nch = rpw // _CH
    ipr = rpw // _CH

    mesh = plsc.VectorSubcoreMesh(core_axis_name="c", subcore_axis_name="s")

    @functools.partial(
        pl.kernel,
        out_type=jax.ShapeDtypeStruct((rows, _ND), jnp.float32),
        mesh=mesh,
        scratch_types=[
            pltpu.VMEM((ipr, _CH), jnp.int32),
            pltpu.VMEM((rpw, _ND), jnp.float32),
            pltpu.VMEM_SHARED((16, _ND), jnp.float32),
            pltpu.SemaphoreType.DMA,
            pltpu.SemaphoreType.DMA,
        ],
        compiler_params=pltpu.CompilerParams(use_tc_tiling_on_sc=False),
    )
    def run(idx_hbm, table_hbm, out_hbm, idx_v, rows_v, tbl_s, gsem, wsem):
        wid = lax.axis_index("s") * _NC + lax.axis_index("c")

        @pl.when(lax.axis_index("s") == 0)
        def _stage_table():
            pltpu.sync_copy(table_hbm, tbl_s)

        pltpu.sync_copy(idx_hbm.at[pl.ds(wid * ipr, ipr)], idx_v)
        plsc.subcore_barrier()
        gathers = [
            pltpu.async_copy(
                tbl_s.at[idx_v.at[j]],
                rows_v.at[pl.ds(j * _CH, _CH)],
                gsem,
            )
            for j in range(nch)
        ]
        writes = []
        for j in range(0, nch, 4):
            for jj in range(j, j + 4):
                gathers[jj].wait()
            writes.append(
                pltpu.async_copy(
                    rows_v.at[pl.ds(j * _CH, 4 * _CH)],
                    out_hbm.at[pl.ds(wid * rpw + j * _CH, 4 * _CH)],
                    wsem,
                )
            )
        for w in writes:
            w.wait()

    return run(idx2d, table)


def kernel(addr_nibbles, nibble_basis):
    b, k = addr_nibbles.shape
    idx2d = addr_nibbles.astype(jnp.int32).reshape(b * k // _CH, _CH)
    out = _encode(idx2d, nibble_basis)
    return out.reshape(b, k * _ND)
